# Initial kernel scaffold; baseline (speedup 1.0000x reference)
import jax
import jax.numpy as jnp
from jax.experimental import pallas as pl


def kernel(x, latents, W1, W2, Wq, bq, gamma, beta, sk1, sk2):
    raise NotImplementedError("write your pallas kernel here")


# speedup vs baseline: 4.8350x; 4.8350x over previous
import functools

import jax
import jax.numpy as jnp
from jax.experimental import pallas as pl

_D_MODEL = 2048
_N_EXPERTS = 64
_TOP_K = 2
_D_QUERY = 128
_N_HEADS = 2
_D_LATENT = 128
_D_INT = 512
_N_SUB = 8
_K_CAND = 2 * _TOP_K

_TILE = 256


def _gelu(v):
    return 0.5 * v * (1.0 + jax.lax.erf(v * (2.0 ** -0.5)))


def _topk_small(s, k):
    n = s.shape[1]
    iota = jax.lax.broadcasted_iota(jnp.int32, s.shape, 1)
    cur = s
    vals, idxs = [], []
    for _ in range(k):
        m = jnp.max(cur, axis=1, keepdims=True)
        am = jnp.min(jnp.where(cur == m, iota, n), axis=1, keepdims=True)
        vals.append(m)
        idxs.append(am)
        cur = jnp.where(iota == am, -jnp.inf, cur)
    return jnp.concatenate(vals, axis=1), jnp.concatenate(idxs, axis=1)


def _p0_kernel(lat_ref, w1_ref, wv_ref, ha_ref, hv_ref):
    ha = _gelu(jnp.dot(lat_ref[...], w1_ref[...],
                       preferred_element_type=jnp.float32))
    ha_ref[...] = ha
    hv_ref[...] = jnp.dot(ha, wv_ref[...], preferred_element_type=jnp.float32)


def _p1_kernel(x_ref, wq_ref, w2u_ref, bq_ref, q_ref, xp_ref, stats_ref):
    xt = x_ref[...]
    q = jax.lax.dot_general(xt, wq_ref[...], (((1,), (1,)), ((), ())),
                            preferred_element_type=jnp.float32) + bq_ref[...]
    q_ref[...] = q
    xp_ref[...] = jax.lax.dot_general(xt, w2u_ref[...], (((1,), (1,)), ((), ())),
                                      preferred_element_type=jnp.float32)
    part = jnp.concatenate(
        [jnp.sum(q, axis=0, keepdims=True),
         jnp.sum(q * q, axis=0, keepdims=True),
         jnp.zeros((6, 2 * _D_QUERY), jnp.float32)], axis=0)

    @pl.when(pl.program_id(0) == 0)
    def _():
        stats_ref[...] = part

    @pl.when(pl.program_id(0) != 0)
    def _():
        stats_ref[...] += part


def _p2_kernel(n_tok, q_ref, xp_ref, stats_ref, gam_ref, bet_ref,
               sk1_ref, sk2_ref, ha_ref, hv_ref, out_ref):
    tile = q_ref.shape[0]
    stats = stats_ref[...]
    mean = stats[0:1, :] * (1.0 / n_tok)
    var = stats[1:2, :] * (1.0 / n_tok) - mean * mean
    rstd = jax.lax.rsqrt(var + 1e-5)
    qn = (q_ref[...] - mean) * (rstd * gam_ref[...]) + bet_ref[...]
    logits = jax.lax.dot_general(xp_ref[...], ha_ref[...],
                                 (((1,), (1,)), ((), ())),
                                 preferred_element_type=jnp.float32)
    iota64 = jax.lax.broadcasted_iota(jnp.int32, (tile, _N_EXPERTS), 1)
    w = jnp.zeros((tile, _N_EXPERTS), jnp.float32)
    for h in range(_N_HEADS):
        qh = qn[:, h * _D_QUERY:(h + 1) * _D_QUERY]
        half = _D_QUERY // 2
        s1 = jax.lax.dot_general(qh[:, :half], sk1_ref[...],
                                 (((1,), (1,)), ((), ())),
                                 preferred_element_type=jnp.float32)
        s2 = jax.lax.dot_general(qh[:, half:], sk2_ref[...],
                                 (((1,), (1,)), ((), ())),
                                 preferred_element_type=jnp.float32)
        ts1, ti1 = _topk_small(s1, _K_CAND)
        ts2, ti2 = _topk_small(s2, _K_CAND)
        comb = jnp.concatenate(
            [ts1[:, i:i + 1] + ts2 for i in range(_K_CAND)], axis=1)
        cv, ci = _topk_small(comb, _TOP_K)
        mx = jnp.max(cv, axis=1, keepdims=True)
        ex = jnp.exp(cv - mx)
        fs = ex / jnp.sum(ex, axis=1, keepdims=True)
        for k in range(_TOP_K):
            cik = ci[:, k:k + 1]
            i1 = cik // _K_CAND
            i2 = cik - i1 * _K_CAND
            e1 = jnp.zeros_like(cik)
            e2 = jnp.zeros_like(cik)
            for j in range(_K_CAND):
                e1 = e1 + jnp.where(i1 == j, ti1[:, j:j + 1], 0)
                e2 = e2 + jnp.where(i2 == j, ti2[:, j:j + 1], 0)
            eidx = e1 * _N_SUB + e2
            mask = eidx == iota64
            lv = jnp.sum(jnp.where(mask, logits, 0.0), axis=1, keepdims=True)
            act = _gelu(lv) * fs[:, k:k + 1]
            w = w + jnp.where(mask, act, 0.0)
    out_ref[...] = jax.lax.dot_general(
        w, hv_ref[...], (((1,), (0,)), ((), ())),
        preferred_element_type=jnp.float32) * (1.0 / _N_HEADS)


def kernel(x, latents, W1, W2, Wq, bq, gamma, beta, sk1, sk2):
    B, S, D = x.shape
    n_tok = B * S
    xf = x.reshape(n_tok, D)
    WqR = Wq.reshape(_N_HEADS * _D_QUERY, D)
    W2u = W2[:, :D]
    Wv = W2[:, D:]
    bqr = bq.reshape(1, -1)
    gam = gamma.reshape(1, -1)
    bet = beta.reshape(1, -1)
    f32 = jnp.float32

    H_all, HV = pl.pallas_call(
        _p0_kernel,
        out_shape=[jax.ShapeDtypeStruct((_N_EXPERTS, _D_INT), f32),
                   jax.ShapeDtypeStruct((_N_EXPERTS, D), f32)],
    )(latents, W1, Wv)

    grid = (n_tok // _TILE,)
    qdim = _N_HEADS * _D_QUERY
    Q, XP, stats = pl.pallas_call(
        _p1_kernel,
        grid=grid,
        in_specs=[
            pl.BlockSpec((_TILE, D), lambda i: (i, 0)),
            pl.BlockSpec((qdim, D), lambda i: (0, 0)),
            pl.BlockSpec((_D_INT, D), lambda i: (0, 0)),
            pl.BlockSpec((1, qdim), lambda i: (0, 0)),
        ],
        out_specs=[
            pl.BlockSpec((_TILE, qdim), lambda i: (i, 0)),
            pl.BlockSpec((_TILE, _D_INT), lambda i: (i, 0)),
            pl.BlockSpec((8, qdim), lambda i: (0, 0)),
        ],
        out_shape=[jax.ShapeDtypeStruct((n_tok, qdim), f32),
                   jax.ShapeDtypeStruct((n_tok, _D_INT), f32),
                   jax.ShapeDtypeStruct((8, qdim), f32)],
    )(xf, WqR, W2u, bqr)

    out = pl.pallas_call(
        functools.partial(_p2_kernel, float(n_tok)),
        grid=grid,
        in_specs=[
            pl.BlockSpec((_TILE, qdim), lambda i: (i, 0)),
            pl.BlockSpec((_TILE, _D_INT), lambda i: (i, 0)),
            pl.BlockSpec((8, qdim), lambda i: (0, 0)),
            pl.BlockSpec((1, qdim), lambda i: (0, 0)),
            pl.BlockSpec((1, qdim), lambda i: (0, 0)),
            pl.BlockSpec((_N_SUB, _D_QUERY // 2), lambda i: (0, 0)),
            pl.BlockSpec((_N_SUB, _D_QUERY // 2), lambda i: (0, 0)),
            pl.BlockSpec((_N_EXPERTS, _D_INT), lambda i: (0, 0)),
            pl.BlockSpec((_N_EXPERTS, D), lambda i: (0, 0)),
        ],
        out_specs=pl.BlockSpec((_TILE, D), lambda i: (i, 0)),
        out_shape=jax.ShapeDtypeStruct((n_tok, D), f32),
    )(Q, XP, stats, gam, bet, sk1, sk2, H_all, HV)

    return out.reshape(B, S, D)


# speedup vs baseline: 4.8929x; 1.0120x over previous
import functools

import jax
import jax.numpy as jnp
from jax.experimental import pallas as pl

_D_MODEL = 2048
_N_EXPERTS = 64
_TOP_K = 2
_D_QUERY = 128
_N_HEADS = 2
_D_LATENT = 128
_D_INT = 512
_N_SUB = 8
_K_CAND = 2 * _TOP_K

_TILE = 256


def _gelu(v):
    return 0.5 * v * (1.0 + jax.lax.erf(v * (2.0 ** -0.5)))


def _topk_small(s, k):
    n = s.shape[1]
    iota = jax.lax.broadcasted_iota(jnp.int32, s.shape, 1)
    cur = s
    vals, idxs = [], []
    for _ in range(k):
        m = jnp.max(cur, axis=1, keepdims=True)
        am = jnp.min(jnp.where(cur == m, iota, n), axis=1, keepdims=True)
        vals.append(m)
        idxs.append(am)
        cur = jnp.where(iota == am, -jnp.inf, cur)
    return jnp.concatenate(vals, axis=1), jnp.concatenate(idxs, axis=1)


def _p0_kernel(lat_ref, w1_ref, wv_ref, w2u_ref, hv_ref, g_ref):
    ha = _gelu(jnp.dot(lat_ref[...], w1_ref[...],
                       preferred_element_type=jnp.float32))
    hv_ref[...] = jnp.dot(ha, wv_ref[...], preferred_element_type=jnp.float32)
    g_ref[...] = jax.lax.dot_general(
        w2u_ref[...], ha, (((0,), (1,)), ((), ())),
        preferred_element_type=jnp.float32)


def _p1_kernel(x_ref, wq_ref, g_ref, bq_ref, q_ref, l_ref, stats_ref):
    xt = x_ref[...]
    q = jax.lax.dot_general(xt, wq_ref[...], (((1,), (1,)), ((), ())),
                            preferred_element_type=jnp.float32) + bq_ref[...]
    q_ref[...] = q
    l_ref[...] = jax.lax.dot_general(xt, g_ref[...], (((1,), (0,)), ((), ())),
                                     preferred_element_type=jnp.float32)
    part = jnp.concatenate(
        [jnp.sum(q, axis=0, keepdims=True),
         jnp.sum(q * q, axis=0, keepdims=True),
         jnp.zeros((6, 2 * _D_QUERY), jnp.float32)], axis=0)

    @pl.when(pl.program_id(0) == 0)
    def _():
        stats_ref[...] = part

    @pl.when(pl.program_id(0) != 0)
    def _():
        stats_ref[...] += part


def _p2_kernel(n_tok, q_ref, l_ref, stats_ref, gam_ref, bet_ref,
               sk1_ref, sk2_ref, hv_ref, out_ref):
    tile = q_ref.shape[0]
    stats = stats_ref[...]
    mean = stats[0:1, :] * (1.0 / n_tok)
    var = stats[1:2, :] * (1.0 / n_tok) - mean * mean
    rstd = jax.lax.rsqrt(var + 1e-5)
    qn = (q_ref[...] - mean) * (rstd * gam_ref[...]) + bet_ref[...]
    logits = l_ref[...]
    iota64 = jax.lax.broadcasted_iota(jnp.int32, (tile, _N_EXPERTS), 1)
    w = jnp.zeros((tile, _N_EXPERTS), jnp.float32)
    for h in range(_N_HEADS):
        qh = qn[:, h * _D_QUERY:(h + 1) * _D_QUERY]
        half = _D_QUERY // 2
        s1 = jax.lax.dot_general(qh[:, :half], sk1_ref[...],
                                 (((1,), (1,)), ((), ())),
                                 preferred_element_type=jnp.float32)
        s2 = jax.lax.dot_general(qh[:, half:], sk2_ref[...],
                                 (((1,), (1,)), ((), ())),
                                 preferred_element_type=jnp.float32)
        ts1, ti1 = _topk_small(s1, _K_CAND)
        ts2, ti2 = _topk_small(s2, _K_CAND)
        comb = jnp.concatenate(
            [ts1[:, i:i + 1] + ts2 for i in range(_K_CAND)], axis=1)
        cv, ci = _topk_small(comb, _TOP_K)
        mx = jnp.max(cv, axis=1, keepdims=True)
        ex = jnp.exp(cv - mx)
        fs = ex / jnp.sum(ex, axis=1, keepdims=True)
        for k in range(_TOP_K):
            cik = ci[:, k:k + 1]
            i1 = cik // _K_CAND
            i2 = cik - i1 * _K_CAND
            e1 = jnp.zeros_like(cik)
            e2 = jnp.zeros_like(cik)
            for j in range(_K_CAND):
                e1 = e1 + jnp.where(i1 == j, ti1[:, j:j + 1], 0)
                e2 = e2 + jnp.where(i2 == j, ti2[:, j:j + 1], 0)
            eidx = e1 * _N_SUB + e2
            mask = eidx == iota64
            lv = jnp.sum(jnp.where(mask, logits, 0.0), axis=1, keepdims=True)
            act = _gelu(lv) * fs[:, k:k + 1]
            w = w + jnp.where(mask, act, 0.0)
    out_ref[...] = jax.lax.dot_general(
        w, hv_ref[...], (((1,), (0,)), ((), ())),
        preferred_element_type=jnp.float32) * (1.0 / _N_HEADS)


def kernel(x, latents, W1, W2, Wq, bq, gamma, beta, sk1, sk2):
    B, S, D = x.shape
    n_tok = B * S
    xf = x.reshape(n_tok, D)
    WqR = Wq.reshape(_N_HEADS * _D_QUERY, D)
    W2u = W2[:, :D]
    Wv = W2[:, D:]
    bqr = bq.reshape(1, -1)
    gam = gamma.reshape(1, -1)
    bet = beta.reshape(1, -1)
    f32 = jnp.float32

    HV, G = pl.pallas_call(
        _p0_kernel,
        out_shape=[jax.ShapeDtypeStruct((_N_EXPERTS, D), f32),
                   jax.ShapeDtypeStruct((D, _N_EXPERTS), f32)],
    )(latents, W1, Wv, W2u)

    grid = (n_tok // _TILE,)
    qdim = _N_HEADS * _D_QUERY
    Q, L, stats = pl.pallas_call(
        _p1_kernel,
        grid=grid,
        in_specs=[
            pl.BlockSpec((_TILE, D), lambda i: (i, 0)),
            pl.BlockSpec((qdim, D), lambda i: (0, 0)),
            pl.BlockSpec((D, _N_EXPERTS), lambda i: (0, 0)),
            pl.BlockSpec((1, qdim), lambda i: (0, 0)),
        ],
        out_specs=[
            pl.BlockSpec((_TILE, qdim), lambda i: (i, 0)),
            pl.BlockSpec((_TILE, _N_EXPERTS), lambda i: (i, 0)),
            pl.BlockSpec((8, qdim), lambda i: (0, 0)),
        ],
        out_shape=[jax.ShapeDtypeStruct((n_tok, qdim), f32),
                   jax.ShapeDtypeStruct((n_tok, _N_EXPERTS), f32),
                   jax.ShapeDtypeStruct((8, qdim), f32)],
    )(xf, WqR, G, bqr)

    out = pl.pallas_call(
        functools.partial(_p2_kernel, float(n_tok)),
        grid=grid,
        in_specs=[
            pl.BlockSpec((_TILE, qdim), lambda i: (i, 0)),
            pl.BlockSpec((_TILE, _N_EXPERTS), lambda i: (i, 0)),
            pl.BlockSpec((8, qdim), lambda i: (0, 0)),
            pl.BlockSpec((1, qdim), lambda i: (0, 0)),
            pl.BlockSpec((1, qdim), lambda i: (0, 0)),
            pl.BlockSpec((_N_SUB, _D_QUERY // 2), lambda i: (0, 0)),
            pl.BlockSpec((_N_SUB, _D_QUERY // 2), lambda i: (0, 0)),
            pl.BlockSpec((_N_EXPERTS, D), lambda i: (0, 0)),
        ],
        out_specs=pl.BlockSpec((_TILE, D), lambda i: (i, 0)),
        out_shape=jax.ShapeDtypeStruct((n_tok, D), f32),
    )(Q, L, stats, gam, bet, sk1, sk2, HV)

    return out.reshape(B, S, D)


# speedup vs baseline: 10.2909x; 2.1032x over previous
import functools

import jax
import jax.numpy as jnp
from jax.experimental import pallas as pl

_D_MODEL = 2048
_N_EXPERTS = 64
_TOP_K = 2
_D_QUERY = 128
_N_HEADS = 2
_D_LATENT = 128
_D_INT = 512
_N_SUB = 8
_K_CAND = 2 * _TOP_K

_TILE = 256


def _gelu(v):
    return 0.5 * v * (1.0 + jax.lax.erf(v * (2.0 ** -0.5)))


def _max_mask(s, iota):
    m = jnp.max(s, axis=1, keepdims=True)
    am = jnp.min(jnp.where(s == m, iota, s.shape[1]), axis=1, keepdims=True)
    return m, iota == am


def _p0_kernel(lat_ref, w1_ref, wv_ref, w2u_ref, hv_ref, g_ref):
    ha = _gelu(jnp.dot(lat_ref[...], w1_ref[...],
                       preferred_element_type=jnp.float32))
    hv_ref[...] = jnp.dot(ha, wv_ref[...], preferred_element_type=jnp.float32)
    g_ref[...] = jax.lax.dot_general(
        w2u_ref[...], ha, (((0,), (1,)), ((), ())),
        preferred_element_type=jnp.float32)


def _p1_kernel(x_ref, wq_ref, g_ref, bq_ref, q_ref, l_ref, stats_ref):
    xt = x_ref[...]
    q = jax.lax.dot_general(xt, wq_ref[...], (((1,), (1,)), ((), ())),
                            preferred_element_type=jnp.float32) + bq_ref[...]
    q_ref[...] = q
    l_ref[...] = jax.lax.dot_general(xt, g_ref[...], (((1,), (0,)), ((), ())),
                                     preferred_element_type=jnp.float32)
    part = jnp.concatenate(
        [jnp.sum(q, axis=0, keepdims=True),
         jnp.sum(q * q, axis=0, keepdims=True),
         jnp.zeros((6, 2 * _D_QUERY), jnp.float32)], axis=0)

    @pl.when(pl.program_id(0) == 0)
    def _():
        stats_ref[...] = part

    @pl.when(pl.program_id(0) != 0)
    def _():
        stats_ref[...] += part


def _p2_kernel(n_tok, q_ref, l_ref, stats_ref, gam_ref, bet_ref,
               skc_ref, hv_ref, out_ref):
    tile = q_ref.shape[0]
    stats = stats_ref[...]
    mean = stats[0:1, :] * (1.0 / n_tok)
    var = stats[1:2, :] * (1.0 / n_tok) - mean * mean
    rstd = jax.lax.rsqrt(var + 1e-5)
    qn = (q_ref[...] - mean) * (rstd * gam_ref[...]) + bet_ref[...]
    glog = _gelu(l_ref[...])
    iota64 = jax.lax.broadcasted_iota(jnp.int32, (tile, _N_EXPERTS), 1)
    w = jnp.zeros((tile, _N_EXPERTS), jnp.float32)
    for h in range(_N_HEADS):
        qh = qn[:, h * _D_QUERY:(h + 1) * _D_QUERY]
        comb = jax.lax.dot_general(qh, skc_ref[...], (((1,), (1,)), ((), ())),
                                   preferred_element_type=jnp.float32)
        m0, mask0 = _max_mask(comb, iota64)
        m1, mask1 = _max_mask(jnp.where(mask0, -jnp.inf, comb), iota64)
        fs0 = 1.0 / (1.0 + jnp.exp(m1 - m0))
        w = w + glog * (jnp.where(mask0, fs0, 0.0)
                        + jnp.where(mask1, 1.0 - fs0, 0.0))
    out_ref[...] = jax.lax.dot_general(
        w, hv_ref[...], (((1,), (0,)), ((), ())),
        preferred_element_type=jnp.float32) * (1.0 / _N_HEADS)


def kernel(x, latents, W1, W2, Wq, bq, gamma, beta, sk1, sk2):
    B, S, D = x.shape
    n_tok = B * S
    xf = x.reshape(n_tok, D)
    WqR = Wq.reshape(_N_HEADS * _D_QUERY, D)
    W2u = W2[:, :D]
    Wv = W2[:, D:]
    bqr = bq.reshape(1, -1)
    gam = gamma.reshape(1, -1)
    bet = beta.reshape(1, -1)
    skcomb = jnp.concatenate(
        [jnp.repeat(sk1, _N_SUB, axis=0), jnp.tile(sk2, (_N_SUB, 1))], axis=1)
    f32 = jnp.float32

    HV, G = pl.pallas_call(
        _p0_kernel,
        out_shape=[jax.ShapeDtypeStruct((_N_EXPERTS, D), f32),
                   jax.ShapeDtypeStruct((D, _N_EXPERTS), f32)],
    )(latents, W1, Wv, W2u)

    grid = (n_tok // _TILE,)
    qdim = _N_HEADS * _D_QUERY
    Q, L, stats = pl.pallas_call(
        _p1_kernel,
        grid=grid,
        in_specs=[
            pl.BlockSpec((_TILE, D), lambda i: (i, 0)),
            pl.BlockSpec((qdim, D), lambda i: (0, 0)),
            pl.BlockSpec((D, _N_EXPERTS), lambda i: (0, 0)),
            pl.BlockSpec((1, qdim), lambda i: (0, 0)),
        ],
        out_specs=[
            pl.BlockSpec((_TILE, qdim), lambda i: (i, 0)),
            pl.BlockSpec((_TILE, _N_EXPERTS), lambda i: (i, 0)),
            pl.BlockSpec((8, qdim), lambda i: (0, 0)),
        ],
        out_shape=[jax.ShapeDtypeStruct((n_tok, qdim), f32),
                   jax.ShapeDtypeStruct((n_tok, _N_EXPERTS), f32),
                   jax.ShapeDtypeStruct((8, qdim), f32)],
    )(xf, WqR, G, bqr)

    out = pl.pallas_call(
        functools.partial(_p2_kernel, float(n_tok)),
        grid=grid,
        in_specs=[
            pl.BlockSpec((_TILE, qdim), lambda i: (i, 0)),
            pl.BlockSpec((_TILE, _N_EXPERTS), lambda i: (i, 0)),
            pl.BlockSpec((8, qdim), lambda i: (0, 0)),
            pl.BlockSpec((1, qdim), lambda i: (0, 0)),
            pl.BlockSpec((1, qdim), lambda i: (0, 0)),
            pl.BlockSpec((_N_EXPERTS, _D_QUERY), lambda i: (0, 0)),
            pl.BlockSpec((_N_EXPERTS, D), lambda i: (0, 0)),
        ],
        out_specs=pl.BlockSpec((_TILE, D), lambda i: (i, 0)),
        out_shape=jax.ShapeDtypeStruct((n_tok, D), f32),
    )(Q, L, stats, gam, bet, skcomb, HV)

    return out.reshape(B, S, D)


# speedup vs baseline: 11.4214x; 1.1099x over previous
import functools

import jax
import jax.numpy as jnp
from jax.experimental import pallas as pl

_D_MODEL = 2048
_N_EXPERTS = 64
_TOP_K = 2
_D_QUERY = 128
_N_HEADS = 2
_D_LATENT = 128
_D_INT = 512
_N_SUB = 8
_K_CAND = 2 * _TOP_K

_TILE = 512


def _gelu(v):
    return 0.5 * v * (1.0 + jax.lax.erf(v * (2.0 ** -0.5)))


def _max_mask(s, iota):
    m = jnp.max(s, axis=1, keepdims=True)
    am = jnp.min(jnp.where(s == m, iota, s.shape[1]), axis=1, keepdims=True)
    return m, iota == am


def _p0_kernel(lat_ref, w1_ref, wv_ref, w2u_ref, hv_ref, g_ref):
    ha = _gelu(jnp.dot(lat_ref[...], w1_ref[...],
                       preferred_element_type=jnp.float32))
    hv_ref[...] = jnp.dot(ha, wv_ref[...], preferred_element_type=jnp.float32)
    g_ref[...] = jax.lax.dot_general(
        w2u_ref[...], ha, (((0,), (1,)), ((), ())),
        preferred_element_type=jnp.float32)


def _p1_kernel(x_ref, wq_ref, g_ref, bq_ref, q_ref, l_ref, stats_ref):
    xt = x_ref[...]
    q = jax.lax.dot_general(xt, wq_ref[...], (((1,), (1,)), ((), ())),
                            preferred_element_type=jnp.float32) + bq_ref[...]
    q_ref[...] = q
    l_ref[...] = jax.lax.dot_general(xt, g_ref[...], (((1,), (0,)), ((), ())),
                                     preferred_element_type=jnp.float32)
    part = jnp.concatenate(
        [jnp.sum(q, axis=0, keepdims=True),
         jnp.sum(q * q, axis=0, keepdims=True),
         jnp.zeros((6, 2 * _D_QUERY), jnp.float32)], axis=0)

    @pl.when(pl.program_id(0) == 0)
    def _():
        stats_ref[...] = part

    @pl.when(pl.program_id(0) != 0)
    def _():
        stats_ref[...] += part


def _p2_kernel(n_tok, q_ref, l_ref, stats_ref, gam_ref, bet_ref,
               skc_ref, hv_ref, out_ref):
    tile = q_ref.shape[0]
    stats = stats_ref[...]
    mean = stats[0:1, :] * (1.0 / n_tok)
    var = stats[1:2, :] * (1.0 / n_tok) - mean * mean
    rstd = jax.lax.rsqrt(var + 1e-5)
    qn = (q_ref[...] - mean) * (rstd * gam_ref[...]) + bet_ref[...]
    glog = _gelu(l_ref[...])
    iota64 = jax.lax.broadcasted_iota(jnp.int32, (tile, _N_EXPERTS), 1)
    w = jnp.zeros((tile, _N_EXPERTS), jnp.float32)
    for h in range(_N_HEADS):
        qh = qn[:, h * _D_QUERY:(h + 1) * _D_QUERY]
        comb = jax.lax.dot_general(qh, skc_ref[...], (((1,), (1,)), ((), ())),
                                   preferred_element_type=jnp.float32)
        m0, mask0 = _max_mask(comb, iota64)
        m1, mask1 = _max_mask(jnp.where(mask0, -jnp.inf, comb), iota64)
        fs0 = 1.0 / (1.0 + jnp.exp(m1 - m0))
        w = w + glog * (jnp.where(mask0, fs0, 0.0)
                        + jnp.where(mask1, 1.0 - fs0, 0.0))
    out_ref[...] = jax.lax.dot_general(
        w, hv_ref[...], (((1,), (0,)), ((), ())),
        preferred_element_type=jnp.float32) * (1.0 / _N_HEADS)


def kernel(x, latents, W1, W2, Wq, bq, gamma, beta, sk1, sk2):
    B, S, D = x.shape
    n_tok = B * S
    xf = x.reshape(n_tok, D)
    WqR = Wq.reshape(_N_HEADS * _D_QUERY, D)
    W2u = W2[:, :D]
    Wv = W2[:, D:]
    bqr = bq.reshape(1, -1)
    gam = gamma.reshape(1, -1)
    bet = beta.reshape(1, -1)
    skcomb = jnp.concatenate(
        [jnp.repeat(sk1, _N_SUB, axis=0), jnp.tile(sk2, (_N_SUB, 1))], axis=1)
    f32 = jnp.float32

    HV, G = pl.pallas_call(
        _p0_kernel,
        out_shape=[jax.ShapeDtypeStruct((_N_EXPERTS, D), f32),
                   jax.ShapeDtypeStruct((D, _N_EXPERTS), f32)],
    )(latents, W1, Wv, W2u)

    grid = (n_tok // _TILE,)
    qdim = _N_HEADS * _D_QUERY
    Q, L, stats = pl.pallas_call(
        _p1_kernel,
        grid=grid,
        in_specs=[
            pl.BlockSpec((_TILE, D), lambda i: (i, 0)),
            pl.BlockSpec((qdim, D), lambda i: (0, 0)),
            pl.BlockSpec((D, _N_EXPERTS), lambda i: (0, 0)),
            pl.BlockSpec((1, qdim), lambda i: (0, 0)),
        ],
        out_specs=[
            pl.BlockSpec((_TILE, qdim), lambda i: (i, 0)),
            pl.BlockSpec((_TILE, _N_EXPERTS), lambda i: (i, 0)),
            pl.BlockSpec((8, qdim), lambda i: (0, 0)),
        ],
        out_shape=[jax.ShapeDtypeStruct((n_tok, qdim), f32),
                   jax.ShapeDtypeStruct((n_tok, _N_EXPERTS), f32),
                   jax.ShapeDtypeStruct((8, qdim), f32)],
    )(xf, WqR, G, bqr)

    out = pl.pallas_call(
        functools.partial(_p2_kernel, float(n_tok)),
        grid=grid,
        in_specs=[
            pl.BlockSpec((_TILE, qdim), lambda i: (i, 0)),
            pl.BlockSpec((_TILE, _N_EXPERTS), lambda i: (i, 0)),
            pl.BlockSpec((8, qdim), lambda i: (0, 0)),
            pl.BlockSpec((1, qdim), lambda i: (0, 0)),
            pl.BlockSpec((1, qdim), lambda i: (0, 0)),
            pl.BlockSpec((_N_EXPERTS, _D_QUERY), lambda i: (0, 0)),
            pl.BlockSpec((_N_EXPERTS, D), lambda i: (0, 0)),
        ],
        out_specs=pl.BlockSpec((_TILE, D), lambda i: (i, 0)),
        out_shape=jax.ShapeDtypeStruct((n_tok, D), f32),
    )(Q, L, stats, gam, bet, skcomb, HV)

    return out.reshape(B, S, D)


# speedup vs baseline: 11.5592x; 1.0121x over previous
import functools

import jax
import jax.numpy as jnp
from jax.experimental import pallas as pl

_D_MODEL = 2048
_N_EXPERTS = 64
_TOP_K = 2
_D_QUERY = 128
_N_HEADS = 2
_D_LATENT = 128
_D_INT = 512
_N_SUB = 8
_K_CAND = 2 * _TOP_K

_TILE = 1024


def _gelu(v):
    return 0.5 * v * (1.0 + jax.lax.erf(v * (2.0 ** -0.5)))


def _max_mask(s, iota):
    m = jnp.max(s, axis=1, keepdims=True)
    am = jnp.min(jnp.where(s == m, iota, s.shape[1]), axis=1, keepdims=True)
    return m, iota == am


def _p0_kernel(lat_ref, w1_ref, wv_ref, w2u_ref, hv_ref, g_ref):
    ha = _gelu(jnp.dot(lat_ref[...], w1_ref[...],
                       preferred_element_type=jnp.float32))
    hv_ref[...] = jnp.dot(ha, wv_ref[...], preferred_element_type=jnp.float32)
    g_ref[...] = jax.lax.dot_general(
        w2u_ref[...], ha, (((0,), (1,)), ((), ())),
        preferred_element_type=jnp.float32)


def _p1_kernel(x_ref, wq_ref, g_ref, bq_ref, q_ref, l_ref, stats_ref):
    xt = x_ref[...]
    q = jax.lax.dot_general(xt, wq_ref[...], (((1,), (1,)), ((), ())),
                            preferred_element_type=jnp.float32) + bq_ref[...]
    q_ref[...] = q
    l_ref[...] = jax.lax.dot_general(xt, g_ref[...], (((1,), (0,)), ((), ())),
                                     preferred_element_type=jnp.float32)
    part = jnp.concatenate(
        [jnp.sum(q, axis=0, keepdims=True),
         jnp.sum(q * q, axis=0, keepdims=True),
         jnp.zeros((6, 2 * _D_QUERY), jnp.float32)], axis=0)

    @pl.when(pl.program_id(0) == 0)
    def _():
        stats_ref[...] = part

    @pl.when(pl.program_id(0) != 0)
    def _():
        stats_ref[...] += part


def _p2_kernel(n_tok, q_ref, l_ref, stats_ref, gam_ref, bet_ref,
               skc_ref, hv_ref, out_ref):
    tile = q_ref.shape[0]
    stats = stats_ref[...]
    mean = stats[0:1, :] * (1.0 / n_tok)
    var = stats[1:2, :] * (1.0 / n_tok) - mean * mean
    rstd = jax.lax.rsqrt(var + 1e-5)
    qn = (q_ref[...] - mean) * (rstd * gam_ref[...]) + bet_ref[...]
    glog = _gelu(l_ref[...])
    iota64 = jax.lax.broadcasted_iota(jnp.int32, (tile, _N_EXPERTS), 1)
    w = jnp.zeros((tile, _N_EXPERTS), jnp.float32)
    for h in range(_N_HEADS):
        qh = qn[:, h * _D_QUERY:(h + 1) * _D_QUERY]
        comb = jax.lax.dot_general(qh, skc_ref[...], (((1,), (1,)), ((), ())),
                                   preferred_element_type=jnp.float32)
        m0, mask0 = _max_mask(comb, iota64)
        m1, mask1 = _max_mask(jnp.where(mask0, -jnp.inf, comb), iota64)
        fs0 = 1.0 / (1.0 + jnp.exp(m1 - m0))
        w = w + glog * (jnp.where(mask0, fs0, 0.0)
                        + jnp.where(mask1, 1.0 - fs0, 0.0))
    out_ref[...] = jax.lax.dot_general(
        w, hv_ref[...], (((1,), (0,)), ((), ())),
        preferred_element_type=jnp.float32) * (1.0 / _N_HEADS)


def kernel(x, latents, W1, W2, Wq, bq, gamma, beta, sk1, sk2):
    B, S, D = x.shape
    n_tok = B * S
    xf = x.reshape(n_tok, D)
    WqR = Wq.reshape(_N_HEADS * _D_QUERY, D)
    W2u = W2[:, :D]
    Wv = W2[:, D:]
    bqr = bq.reshape(1, -1)
    gam = gamma.reshape(1, -1)
    bet = beta.reshape(1, -1)
    skcomb = jnp.concatenate(
        [jnp.repeat(sk1, _N_SUB, axis=0), jnp.tile(sk2, (_N_SUB, 1))], axis=1)
    f32 = jnp.float32

    HV, G = pl.pallas_call(
        _p0_kernel,
        out_shape=[jax.ShapeDtypeStruct((_N_EXPERTS, D), f32),
                   jax.ShapeDtypeStruct((D, _N_EXPERTS), f32)],
    )(latents, W1, Wv, W2u)

    grid = (n_tok // _TILE,)
    qdim = _N_HEADS * _D_QUERY
    Q, L, stats = pl.pallas_call(
        _p1_kernel,
        grid=grid,
        in_specs=[
            pl.BlockSpec((_TILE, D), lambda i: (i, 0)),
            pl.BlockSpec((qdim, D), lambda i: (0, 0)),
            pl.BlockSpec((D, _N_EXPERTS), lambda i: (0, 0)),
            pl.BlockSpec((1, qdim), lambda i: (0, 0)),
        ],
        out_specs=[
            pl.BlockSpec((_TILE, qdim), lambda i: (i, 0)),
            pl.BlockSpec((_TILE, _N_EXPERTS), lambda i: (i, 0)),
            pl.BlockSpec((8, qdim), lambda i: (0, 0)),
        ],
        out_shape=[jax.ShapeDtypeStruct((n_tok, qdim), f32),
                   jax.ShapeDtypeStruct((n_tok, _N_EXPERTS), f32),
                   jax.ShapeDtypeStruct((8, qdim), f32)],
    )(xf, WqR, G, bqr)

    out = pl.pallas_call(
        functools.partial(_p2_kernel, float(n_tok)),
        grid=grid,
        in_specs=[
            pl.BlockSpec((_TILE, qdim), lambda i: (i, 0)),
            pl.BlockSpec((_TILE, _N_EXPERTS), lambda i: (i, 0)),
            pl.BlockSpec((8, qdim), lambda i: (0, 0)),
            pl.BlockSpec((1, qdim), lambda i: (0, 0)),
            pl.BlockSpec((1, qdim), lambda i: (0, 0)),
            pl.BlockSpec((_N_EXPERTS, _D_QUERY), lambda i: (0, 0)),
            pl.BlockSpec((_N_EXPERTS, D), lambda i: (0, 0)),
        ],
        out_specs=pl.BlockSpec((_TILE, D), lambda i: (i, 0)),
        out_shape=jax.ShapeDtypeStruct((n_tok, D), f32),
    )(Q, L, stats, gam, bet, skcomb, HV)

    return out.reshape(B, S, D)


# speedup vs baseline: 13.5546x; 1.1726x over previous
import jax
import jax.numpy as jnp
from jax.experimental import pallas as pl
from jax.experimental.pallas import tpu as pltpu

_D_MODEL = 2048
_N_EXPERTS = 64
_D_QUERY = 128
_N_HEADS = 2
_D_INT = 512
_N_SUB = 8

_TILE = 512
_N_TOK = 2048
_N_TILES = _N_TOK // _TILE
_QDIM = _N_HEADS * _D_QUERY


def _gelu(v):
    return 0.5 * v * (1.0 + jax.lax.erf(v * (2.0 ** -0.5)))


def _max_mask(s, iota):
    m = jnp.max(s, axis=1, keepdims=True)
    am = jnp.min(jnp.where(s == m, iota, s.shape[1]), axis=1, keepdims=True)
    return m, iota == am


def _fused_kernel(x_ref, wq_ref, lat_ref, w1_ref, w2u_ref, wv_ref, bq_ref,
                  gam_ref, bet_ref, skc_ref, out_ref,
                  q_s, l_s, g_s, hv_s, stats_s):
    i = pl.program_id(0)

    @pl.when(i == 0)
    def _prep():
        ha = _gelu(jnp.dot(lat_ref[...], w1_ref[...],
                           preferred_element_type=jnp.float32))
        hv_s[...] = jnp.dot(ha, wv_ref[...],
                            preferred_element_type=jnp.float32)
        g_s[...] = jax.lax.dot_general(
            w2u_ref[...], ha, (((0,), (1,)), ((), ())),
            preferred_element_type=jnp.float32)

    @pl.when(i < _N_TILES)
    def _phase_a():
        xt = x_ref[...]
        q = jax.lax.dot_general(xt, wq_ref[...], (((1,), (1,)), ((), ())),
                                preferred_element_type=jnp.float32) + bq_ref[...]
        q_s[pl.ds(i * _TILE, _TILE), :] = q
        l_s[pl.ds(i * _TILE, _TILE), :] = jax.lax.dot_general(
            xt, g_s[...], (((1,), (0,)), ((), ())),
            preferred_element_type=jnp.float32)
        part = jnp.concatenate(
            [jnp.sum(q, axis=0, keepdims=True),
             jnp.sum(q * q, axis=0, keepdims=True),
             jnp.zeros((6, _QDIM), jnp.float32)], axis=0)

        @pl.when(i == 0)
        def _():
            stats_s[...] = part

        @pl.when(i != 0)
        def _():
            stats_s[...] += part

    @pl.when(i >= _N_TILES)
    def _phase_b():
        j = i - _N_TILES
        stats = stats_s[...]
        mean = stats[0:1, :] * (1.0 / _N_TOK)
        var = stats[1:2, :] * (1.0 / _N_TOK) - mean * mean
        rstd = jax.lax.rsqrt(var + 1e-5)
        qn = ((q_s[pl.ds(j * _TILE, _TILE), :] - mean)
              * (rstd * gam_ref[...]) + bet_ref[...])
        glog = _gelu(l_s[pl.ds(j * _TILE, _TILE), :])
        iota64 = jax.lax.broadcasted_iota(jnp.int32, (_TILE, _N_EXPERTS), 1)
        w = jnp.zeros((_TILE, _N_EXPERTS), jnp.float32)
        for h in range(_N_HEADS):
            qh = qn[:, h * _D_QUERY:(h + 1) * _D_QUERY]
            comb = jax.lax.dot_general(qh, skc_ref[...],
                                       (((1,), (1,)), ((), ())),
                                       preferred_element_type=jnp.float32)
            m0, mask0 = _max_mask(comb, iota64)
            m1, mask1 = _max_mask(jnp.where(mask0, -jnp.inf, comb), iota64)
            fs0 = 1.0 / (1.0 + jnp.exp(m1 - m0))
            w = w + glog * (jnp.where(mask0, fs0, 0.0)
                            + jnp.where(mask1, 1.0 - fs0, 0.0))
        out_ref[...] = jax.lax.dot_general(
            w, hv_s[...], (((1,), (0,)), ((), ())),
            preferred_element_type=jnp.float32) * (1.0 / _N_HEADS)


def kernel(x, latents, W1, W2, Wq, bq, gamma, beta, sk1, sk2):
    B, S, D = x.shape
    n_tok = B * S
    xf = x.reshape(n_tok, D)
    WqR = Wq.reshape(_QDIM, D)
    W2u = W2[:, :D]
    Wv = W2[:, D:]
    bqr = bq.reshape(1, -1)
    gam = gamma.reshape(1, -1)
    bet = beta.reshape(1, -1)
    skcomb = jnp.concatenate(
        [jnp.repeat(sk1, _N_SUB, axis=0), jnp.tile(sk2, (_N_SUB, 1))], axis=1)
    f32 = jnp.float32

    nt = _N_TILES
    out = pl.pallas_call(
        _fused_kernel,
        grid=(2 * nt,),
        in_specs=[
            pl.BlockSpec((_TILE, D), lambda i: (jnp.minimum(i, nt - 1), 0)),
            pl.BlockSpec((_QDIM, D), lambda i: (0, 0)),
            pl.BlockSpec((_N_EXPERTS, _D_QUERY), lambda i: (0, 0)),
            pl.BlockSpec((_D_QUERY, _D_INT), lambda i: (0, 0)),
            pl.BlockSpec((_D_INT, D), lambda i: (0, 0)),
            pl.BlockSpec((_D_INT, D), lambda i: (0, 0)),
            pl.BlockSpec((1, _QDIM), lambda i: (0, 0)),
            pl.BlockSpec((1, _QDIM), lambda i: (0, 0)),
            pl.BlockSpec((1, _QDIM), lambda i: (0, 0)),
            pl.BlockSpec((_N_EXPERTS, _D_QUERY), lambda i: (0, 0)),
        ],
        out_specs=pl.BlockSpec(
            (_TILE, D), lambda i: (jnp.maximum(i - nt, 0), 0)),
        out_shape=jax.ShapeDtypeStruct((n_tok, D), f32),
        scratch_shapes=[
            pltpu.VMEM((_N_TOK, _QDIM), f32),
            pltpu.VMEM((_N_TOK, _N_EXPERTS), f32),
            pltpu.VMEM((D, _N_EXPERTS), f32),
            pltpu.VMEM((_N_EXPERTS, D), f32),
            pltpu.VMEM((8, _QDIM), f32),
        ],
    )(xf, WqR, latents, W1, W2u, Wv, bqr, gam, bet, skcomb)

    return out.reshape(B, S, D)
